# Initial kernel scaffold; baseline (speedup 1.0000x reference)
import jax
import jax.numpy as jnp
from jax.experimental import pallas as pl


def kernel(inputs, widths, aa_factors):
    raise NotImplementedError("write your pallas kernel here")


# speedup vs baseline: 2.2293x; 2.2293x over previous
import functools
from math import comb

import jax
import jax.numpy as jnp
import numpy as np
from jax import lax
from jax.experimental import pallas as pl
from jax.experimental.pallas import tpu as pltpu

_H, _W = 224, 224
_S = 15
_K = 4
_EPS = 1e-06


def _basis() -> jnp.ndarray:
    ts = jnp.linspace(0.0, 1.0, _S)
    i = np.arange(_K)
    coeff = np.array([comb(_K - 1, j) for j in range(_K)], dtype=np.float32)
    return (coeff[None, :] * (ts[:, None] ** i[None, :])
            * ((1.0 - ts[:, None]) ** (_K - 1 - i[None, :]))).astype(jnp.float32)


def _curve_kernel(sy_ref, sx_ref, syq_ref, sxq_ref, w_ref, aa_ref, out_ref):
    b = pl.program_id(0)

    yf = lax.broadcasted_iota(jnp.int32, (_H, _W), 0).astype(jnp.float32)
    xf = lax.broadcasted_iota(jnp.int32, (_H, _W), 1).astype(jnp.float32)
    p2 = yf * yf + xf * xf

    m = None
    for s in range(_S):
        dot = yf * syq_ref[b, s] + xf * sxq_ref[b, s]
        s2 = sy_ref[b, s] * sy_ref[b, s] + sx_ref[b, s] * sx_ref[b, s]
        d2 = (p2 - 2.0 * dot) + s2
        m = d2 if m is None else jnp.minimum(m, d2)

    dmin = jnp.sqrt(jnp.maximum(m, 0.0) + 1e-12)
    w = w_ref[b]
    aa = aa_ref[b]
    val = 1.0 - (dmin / w + _EPS) ** aa
    out_ref[0] = jnp.where(dmin < w, val, 0.0)


@jax.jit
def kernel(inputs, widths, aa_factors):
    B = inputs.shape[0]
    kp = inputs * jnp.array([float(_H), float(_W)], dtype=jnp.float32)
    sp = jnp.einsum('sk,bkd->bsd', _basis(), kp)
    sy = sp[:, :, 0]
    sx = sp[:, :, 1]
    def _rne_bf16(x):
        u = lax.bitcast_convert_type(x, jnp.uint32)
        u = u + jnp.uint32(0x7FFF) + ((u >> 16) & jnp.uint32(1))
        return lax.bitcast_convert_type(u & jnp.uint32(0xFFFF0000), jnp.float32)

    syq = _rne_bf16(sy)
    sxq = _rne_bf16(sx)

    return pl.pallas_call(
        _curve_kernel,
        grid=(B,),
        in_specs=[pl.BlockSpec(memory_space=pltpu.SMEM)] * 6,
        out_specs=pl.BlockSpec((1, _H, _W), lambda b: (b, 0, 0)),
        out_shape=jax.ShapeDtypeStruct((B, _H, _W), jnp.float32),
    )(sy, sx, syq, sxq, widths, aa_factors)
